# baseline (speedup 1.0000x reference)
import functools

import jax
import jax.numpy as jnp
from jax import lax
from jax.experimental import pallas as pl
from jax.experimental.pallas import tpu as pltpu
from jax.experimental.pallas import tpu_sc as plsc

NC = 2
NS = 16
NW = NC * NS
LANES = 128
GBUF = 2


@functools.partial(jax.jit, static_argnums=(2, 3))
def _emb_lookup(w128, idx_t, n_s, d):
    mesh = plsc.VectorSubcoreMesh(core_axis_name="c", subcore_axis_name="s")
    n_g = d // 8

    @functools.partial(
        pl.kernel,
        mesh=mesh,
        out_type=jax.ShapeDtypeStruct((n_s * n_g * NW, 8, LANES), jnp.float32),
        compiler_params=pltpu.CompilerParams(
            use_tc_tiling_on_sc=False, needs_layout_passes=False
        ),
        scratch_types=[
            pltpu.VMEM((n_s, LANES), jnp.int32),
            pltpu.VMEM((GBUF, LANES, LANES), jnp.float32),
            pltpu.VMEM((GBUF, n_g, 8, LANES), jnp.float32),
            pltpu.SemaphoreType.DMA((GBUF,)),
            pltpu.SemaphoreType.DMA((GBUF,)),
        ],
    )
    def body(table_hbm, idx_hbm, p_hbm, idx_v, g_v, p_v, gsem, osem):
        wid = lax.axis_index("s") * NC + lax.axis_index("c")
        pltpu.sync_copy(idx_hbm.at[:, pl.ds(wid * LANES, LANES)], idx_v)

        def gather_descr(s, buf):
            return pltpu.make_async_copy(
                table_hbm.at[idx_v.at[s]], g_v.at[buf], gsem.at[buf]
            )

        def out_descr(s, g, buf):
            t = (s * 8 + g) * NW + wid
            return pltpu.make_async_copy(
                p_v.at[buf, g], p_hbm.at[t], osem.at[buf]
            )

        rows_vecs = [
            lax.iota(jnp.int32, 16) + (16 * m) for m in range(LANES // 16)
        ]

        def transpose_task(buf):
            for m in range(LANES // 16):
                rows = rows_vecs[m]
                for f in range(8 * n_g):
                    cols = jnp.full((16,), f, jnp.int32)
                    vals = plsc.load_gather(g_v.at[buf], [rows, cols])
                    p_v[buf, f // 8, f % 8, pl.ds(16 * m, 16)] = vals

        for b in range(GBUF):
            gather_descr(b, b).start()

        @pl.loop(0, n_s, step=GBUF)
        def _(s4):
            for k in range(GBUF):
                s = s4 + k
                gather_descr(s, k).wait()

                @pl.when(s >= GBUF)
                def _():
                    for g in range(n_g):
                        out_descr(s - GBUF, g, k).wait()

                transpose_task(k)
                for g in range(n_g):
                    out_descr(s, g, k).start()

                @pl.when(s + GBUF < n_s)
                def _():
                    gather_descr(s + GBUF, k).start()

        for k in range(GBUF):
            for g in range(n_g):
                out_descr(n_s - GBUF + k, g, k).wait()

    return body(w128, idx_t)


def kernel(input, weight):
    s0, s1 = input.shape
    v, d = weight.shape
    w128 = jnp.concatenate(
        [weight, jnp.zeros((v, LANES - d), jnp.float32)], axis=1
    )
    idx_t = input.T.astype(jnp.int32)
    p = _emb_lookup(w128, idx_t, s1, d)
    out = (
        p.reshape(s1, d // 8, s0 // LANES, 8, LANES)
        .transpose(2, 4, 0, 1, 3)
        .reshape(s0, s1, d)
    )
    return out


# speedup vs baseline: 1.2048x; 1.2048x over previous
import functools

import jax
import jax.numpy as jnp
from jax import lax
from jax.experimental import pallas as pl
from jax.experimental.pallas import tpu as pltpu
from jax.experimental.pallas import tpu_sc as plsc

NC = 2
NS = 16
NW = NC * NS
LANES = 128
GBUF = 2


@functools.partial(jax.jit, static_argnums=(2, 3))
def _emb_lookup(w128, idx_t, n_s, d):
    mesh = plsc.VectorSubcoreMesh(core_axis_name="c", subcore_axis_name="s")
    n_g = d // 8

    @functools.partial(
        pl.kernel,
        mesh=mesh,
        out_type=jax.ShapeDtypeStruct((n_s * n_g * NW, 8, LANES), jnp.float32),
        compiler_params=pltpu.CompilerParams(
            use_tc_tiling_on_sc=False, needs_layout_passes=False
        ),
        scratch_types=[
            pltpu.VMEM((n_s, LANES), jnp.int32),
            pltpu.VMEM((GBUF, LANES, LANES), jnp.float32),
            pltpu.VMEM((GBUF, n_g, 8, LANES), jnp.float32),
            pltpu.SemaphoreType.DMA((GBUF,)),
            pltpu.SemaphoreType.DMA((GBUF,)),
        ],
    )
    def body(table_hbm, idx_hbm, p_hbm, idx_v, g_v, p_v, gsem, osem):
        wid = lax.axis_index("s") * NC + lax.axis_index("c")
        pltpu.sync_copy(idx_hbm.at[:, pl.ds(wid * LANES, LANES)], idx_v)

        def gather_descr(s, buf):
            return pltpu.make_async_copy(
                table_hbm.at[idx_v.at[s]], g_v.at[buf], gsem.at[buf]
            )

        def out_descr(s, g, buf):
            t = (s * 8 + g) * NW + wid
            return pltpu.make_async_copy(
                p_v.at[buf, g], p_hbm.at[t], osem.at[buf]
            )

        rows_vecs = [
            lax.iota(jnp.int32, 16) + (16 * m) for m in range(LANES // 16)
        ]

        def transpose_task(buf):
            depth = 6
            seq = [(f, m) for f in range(8 * n_g) for m in range(LANES // 16)]
            pending = {}

            def flush(i):
                v, f2, m2 = pending.pop(i)
                p_v[buf, f2 // 8, f2 % 8, pl.ds(16 * m2, 16)] = v

            for i, (f, m) in enumerate(seq):
                cols = jnp.full((16,), f, jnp.int32)
                rows = lax.iota(jnp.int32, 16) + 16 * m
                pending[i] = (
                    plsc.load_gather(g_v.at[buf], [rows, cols]), f, m
                )
                if i >= depth:
                    flush(i - depth)
            for i in sorted(pending):
                flush(i)

        for b in range(GBUF):
            gather_descr(b, b).start()

        @pl.loop(0, n_s, step=GBUF)
        def _(s4):
            for k in range(GBUF):
                s = s4 + k
                gather_descr(s, k).wait()

                @pl.when(s >= GBUF)
                def _():
                    for g in range(n_g):
                        out_descr(s - GBUF, g, k).wait()

                transpose_task(k)
                for g in range(n_g):
                    out_descr(s, g, k).start()

                @pl.when(s + GBUF < n_s)
                def _():
                    gather_descr(s + GBUF, k).start()

        for k in range(GBUF):
            for g in range(n_g):
                out_descr(n_s - GBUF + k, g, k).wait()

    return body(w128, idx_t)


def kernel(input, weight):
    s0, s1 = input.shape
    v, d = weight.shape
    w128 = jnp.concatenate(
        [weight, jnp.zeros((v, LANES - d), jnp.float32)], axis=1
    )
    idx_t = input.T.astype(jnp.int32)
    p = _emb_lookup(w128, idx_t, s1, d)
    out = (
        p.reshape(s1, d // 8, s0 // LANES, 8, LANES)
        .transpose(2, 4, 0, 1, 3)
        .reshape(s0, s1, d)
    )
    return out
